# baseline (speedup 1.0000x reference)
import functools

import jax
import jax.numpy as jnp
from jax import lax
from jax.experimental import pallas as pl
from jax.experimental.pallas import tpu as pltpu
from jax.experimental.pallas import tpu_sc as plsc

NUM_EMB = 1000000
DIM = 64
BATCH = 4096
SEQ = 200

BBLK = 128
NW = 32
NBUF = 4
L = 16


def _gather_sc(tok_t, table2):
    mesh = plsc.VectorSubcoreMesh(core_axis_name="c", subcore_axis_name="s")

    @functools.partial(
        pl.kernel,
        mesh=mesh,
        out_type=jax.ShapeDtypeStruct((SEQ, DIM, BATCH), jnp.float32),
        compiler_params=pltpu.CompilerParams(
            use_tc_tiling_on_sc=True, needs_layout_passes=False
        ),
        scratch_types=(
            [pltpu.VMEM((SEQ, BBLK), jnp.int32)]
            + [pltpu.VMEM((NBUF, BBLK), jnp.int32)]
            + [pltpu.VMEM((NBUF, BBLK), jnp.int32)]
            + [pltpu.VMEM((BBLK, BBLK), jnp.float32) for _ in range(NBUF)]
            + [pltpu.VMEM((DIM, BBLK), jnp.float32) for _ in range(2)]
            + [pltpu.SemaphoreType.DMA for _ in range(NBUF)]
            + [pltpu.SemaphoreType.DMA for _ in range(2)]
        ),
    )
    def body(tok_hbm, table_hbm, out_hbm, tokv, idxv, offv, *rest):
        dstb = rest[:NBUF]
        outb = rest[NBUF:NBUF + 2]
        gsems = rest[NBUF + 2:2 * NBUF + 2]
        osems = rest[2 * NBUF + 2:]
        wid = lax.axis_index("s") * 2 + lax.axis_index("c")
        b0 = wid * BBLK

        pltpu.sync_copy(tok_hbm.at[:, pl.ds(b0, BBLK)], tokv)

        def prep_and_fire(s, b):
            for g in range(BBLK // L):
                t = tokv[s, pl.ds(g * L, L)]
                idxv[b, pl.ds(g * L, L)] = lax.shift_right_logical(t, 1)
                offv[b, pl.ds(g * L, L)] = lax.shift_left(
                    lax.bitwise_and(t, 1), 6
                )
            pltpu.async_copy(table_hbm.at[idxv.at[b]], dstb[b], gsems[b])

        for b in range(NBUF):
            prep_and_fire(b, b)

        jv = [lax.iota(jnp.int32, L) + g * L for g in range(BBLK // L)]

        def outer(i, carry):
            base = i * NBUF
            for b in range(NBUF):
                s = base + b
                o = b % 2
                pltpu.make_async_copy(
                    table_hbm.at[idxv.at[b]], dstb[b], gsems[b]
                ).wait()
                @pl.when(s >= 2)
                def _():
                    pltpu.make_async_copy(
                        outb[o], out_hbm.at[s, :, pl.ds(b0, BBLK)], osems[o]
                    ).wait()

                def transpose_d(d, carry2):
                    for g in range(BBLK // L):
                        cv = offv[b, pl.ds(g * L, L)] + d
                        outb[o][d, pl.ds(g * L, L)] = plsc.load_gather(
                            dstb[b], [jv[g], cv]
                        )
                    return carry2

                lax.fori_loop(0, DIM, transpose_d, 0)
                pltpu.async_copy(
                    outb[o], out_hbm.at[s, :, pl.ds(b0, BBLK)], osems[o]
                )

                @pl.when(s + NBUF < SEQ)
                def _():
                    prep_and_fire(s + NBUF, b)

            return carry

        lax.fori_loop(0, SEQ // NBUF, outer, 0)

        for s in (SEQ - 2, SEQ - 1):
            o = s % NBUF % 2
            pltpu.make_async_copy(
                outb[o], out_hbm.at[s, :, pl.ds(b0, BBLK)], osems[o]
            ).wait()

    return body(tok_t, table2)


def kernel(token_ids, embeddings):
    tok_t = token_ids.T.astype(jnp.int32)
    table2 = embeddings.reshape(NUM_EMB // 2, 2 * DIM)
    out_t = _gather_sc(tok_t, table2)
    return out_t.transpose(2, 0, 1)


# speedup vs baseline: 1.9911x; 1.9911x over previous
import functools

import jax
import jax.numpy as jnp
from jax import lax
from jax.experimental import pallas as pl
from jax.experimental.pallas import tpu as pltpu
from jax.experimental.pallas import tpu_sc as plsc

NUM_EMB = 1000000
DIM = 64
BATCH = 4096
SEQ = 200

BBLK = 128
NW = 32
NBUF = 2
L = 16
NG = BBLK // L


def _gather_sc(tok_t, table2):
    mesh = plsc.VectorSubcoreMesh(core_axis_name="c", subcore_axis_name="s")

    @functools.partial(
        pl.kernel,
        mesh=mesh,
        out_type=jax.ShapeDtypeStruct((SEQ, DIM, BATCH), jnp.float32),
        compiler_params=pltpu.CompilerParams(
            use_tc_tiling_on_sc=True, needs_layout_passes=False
        ),
        scratch_types=(
            [pltpu.VMEM((SEQ, BBLK), jnp.int32)]
            + [pltpu.VMEM((NBUF, BBLK), jnp.int32)]
            + [pltpu.VMEM((NBUF, BBLK), jnp.int32)]
            + [pltpu.VMEM((BBLK, BBLK), jnp.float32) for _ in range(NBUF)]
            + [pltpu.VMEM((DIM, BBLK), jnp.float32) for _ in range(NBUF)]
            + [pltpu.SemaphoreType.DMA for _ in range(NBUF)]
            + [pltpu.SemaphoreType.DMA for _ in range(NBUF)]
        ),
    )
    def body(tok_hbm, table_hbm, out_hbm, tokv, idxv, offv, *rest):
        dstb = rest[:NBUF]
        outb = rest[NBUF:2 * NBUF]
        gsems = rest[2 * NBUF:3 * NBUF]
        osems = rest[3 * NBUF:]
        wid = lax.axis_index("s") * 2 + lax.axis_index("c")
        b0 = wid * BBLK

        pltpu.sync_copy(tok_hbm.at[:, pl.ds(b0, BBLK)], tokv)

        def prep_and_fire(s, b):
            for g in range(NG):
                t = tokv[s, pl.ds(g * L, L)]
                idxv[b, pl.ds(g * L, L)] = lax.shift_right_logical(t, 1)
                offv[b, pl.ds(g * L, L)] = lax.shift_left(
                    lax.bitwise_and(t, 1), 6
                )
            pltpu.async_copy(table_hbm.at[idxv.at[b]], dstb[b], gsems[b])

        for b in range(NBUF):
            prep_and_fire(b, b)

        jv = [lax.iota(jnp.int32, L) + g * L for g in range(NG)]

        def outer(i, carry):
            base = i * NBUF
            for b in range(NBUF):
                s = base + b
                pltpu.make_async_copy(
                    table_hbm.at[idxv.at[b]], dstb[b], gsems[b]
                ).wait()
                @pl.when(s >= NBUF)
                def _():
                    pltpu.make_async_copy(
                        outb[b], out_hbm.at[s, :, pl.ds(b0, BBLK)], osems[b]
                    ).wait()

                for g in range(NG):
                    offg = offv[b, pl.ds(g * L, L)]
                    jvg = jv[g]

                    @plsc.parallel_loop(0, DIM, step=1, unroll=8)
                    def _(d, _g=g, _b=b, _offg=offg, _jvg=jvg):
                        outb[_b][d, pl.ds(_g * L, L)] = plsc.load_gather(
                            dstb[_b], [_jvg, _offg + d]
                        )
                pltpu.async_copy(
                    outb[b], out_hbm.at[s, :, pl.ds(b0, BBLK)], osems[b]
                )

                @pl.when(s + NBUF < SEQ)
                def _():
                    prep_and_fire(s + NBUF, b)

            return carry

        lax.fori_loop(0, SEQ // NBUF, outer, 0)

        for k in range(NBUF):
            s = SEQ - NBUF + k
            b = s % NBUF
            pltpu.make_async_copy(
                outb[b], out_hbm.at[s, :, pl.ds(b0, BBLK)], osems[b]
            ).wait()

    return body(tok_t, table2)


def kernel(token_ids, embeddings):
    tok_t = token_ids.T.astype(jnp.int32)
    table2 = embeddings.reshape(NUM_EMB // 2, 2 * DIM)
    out_t = _gather_sc(tok_t, table2)
    return out_t.transpose(2, 0, 1)


# speedup vs baseline: 2.8754x; 1.4441x over previous
import functools

import jax
import jax.numpy as jnp
from jax import lax
from jax.experimental import pallas as pl
from jax.experimental.pallas import tpu as pltpu
from jax.experimental.pallas import tpu_sc as plsc

NUM_EMB = 1000000
DIM = 64
BATCH = 4096
SEQ = 200

BBLK = 128
NW = 32
NBUF = 4
L = 16
NG = BBLK // L
ND = DIM // L


def _gather_sc(tok_t, table2):
    mesh = plsc.VectorSubcoreMesh(core_axis_name="c", subcore_axis_name="s")

    @functools.partial(
        pl.kernel,
        mesh=mesh,
        out_type=jax.ShapeDtypeStruct((SEQ, DIM, BATCH), jnp.float32),
        compiler_params=pltpu.CompilerParams(
            use_tc_tiling_on_sc=True, needs_layout_passes=False
        ),
        scratch_types=(
            [pltpu.VMEM((SEQ, BBLK), jnp.int32)]
            + [pltpu.VMEM((NBUF, BBLK), jnp.int32)]
            + [pltpu.VMEM((NBUF, BBLK), jnp.int32)]
            + [pltpu.VMEM((BBLK, BBLK), jnp.float32) for _ in range(NBUF)]
            + [pltpu.VMEM((DIM, BBLK), jnp.float32) for _ in range(NBUF)]
            + [pltpu.SemaphoreType.DMA for _ in range(NBUF)]
            + [pltpu.SemaphoreType.DMA for _ in range(NBUF)]
        ),
    )
    def body(tok_hbm, table_hbm, out_hbm, tokv, idxv, offv, *rest):
        dstb = rest[:NBUF]
        outb = rest[NBUF:2 * NBUF]
        gsems = rest[2 * NBUF:3 * NBUF]
        osems = rest[3 * NBUF:]
        wid = lax.axis_index("s") * 2 + lax.axis_index("c")
        b0 = wid * BBLK

        pltpu.sync_copy(tok_hbm.at[:, pl.ds(b0, BBLK)], tokv)

        def prep_and_fire(s, b):
            for g in range(NG):
                t = tokv[s, pl.ds(g * L, L)]
                idxv[b, pl.ds(g * L, L)] = lax.shift_right_logical(t, 1)
                offv[b, pl.ds(g * L, L)] = lax.shift_left(
                    lax.bitwise_and(t, 1), 6
                )
            pltpu.async_copy(table_hbm.at[idxv.at[b]], dstb[b], gsems[b])

        for b in range(NBUF):
            prep_and_fire(b, b)

        iot = lax.iota(jnp.int32, L)
        jv = [iot + g * L for g in range(NG)]

        def outer(i, carry):
            base = i * NBUF
            for b in range(NBUF):
                s = base + b
                pltpu.make_async_copy(
                    table_hbm.at[idxv.at[b]], dstb[b], gsems[b]
                ).wait()
                @pl.when(s >= NBUF)
                def _():
                    pltpu.make_async_copy(
                        outb[b], out_hbm.at[s, :, pl.ds(b0, BBLK)], osems[b]
                    ).wait()

                @plsc.parallel_loop(0, L, step=1, unroll=4)
                def _(c, _b=b):
                    dmod = lax.bitwise_and(iot + c, L - 1)
                    for g in range(NG):
                        colg = offv[_b, pl.ds(g * L, L)] + dmod
                        for k in range(ND):
                            d0 = k * L
                            plsc.store_scatter(
                                outb[_b],
                                [dmod + d0, jv[g]],
                                plsc.load_gather(
                                    dstb[_b], [jv[g], colg + d0]
                                ),
                            )

                pltpu.async_copy(
                    outb[b], out_hbm.at[s, :, pl.ds(b0, BBLK)], osems[b]
                )

                @pl.when(s + NBUF < SEQ)
                def _():
                    prep_and_fire(s + NBUF, b)

            return carry

        lax.fori_loop(0, SEQ // NBUF, outer, 0)

        for k in range(NBUF):
            s = SEQ - NBUF + k
            b = s % NBUF
            pltpu.make_async_copy(
                outb[b], out_hbm.at[s, :, pl.ds(b0, BBLK)], osems[b]
            ).wait()

    return body(tok_t, table2)


def kernel(token_ids, embeddings):
    tok_t = token_ids.T.astype(jnp.int32)
    table2 = embeddings.reshape(NUM_EMB // 2, 2 * DIM)
    out_t = _gather_sc(tok_t, table2)
    return out_t.transpose(2, 0, 1)


# speedup vs baseline: 4.7545x; 1.6535x over previous
import functools

import jax
import jax.numpy as jnp
from jax import lax
from jax.experimental import pallas as pl
from jax.experimental.pallas import tpu as pltpu
from jax.experimental.pallas import tpu_sc as plsc

NUM_EMB = 1000000
DIM = 64
BATCH = 4096
SEQ = 200

BBLK = 128
NW = 32
NBUF = 4
L = 16
NG = BBLK // L
ND = DIM // L


def _gather_sc(tok_t, table3):
    mesh = plsc.VectorSubcoreMesh(core_axis_name="c", subcore_axis_name="s")

    @functools.partial(
        pl.kernel,
        mesh=mesh,
        out_type=jax.ShapeDtypeStruct((SEQ, DIM, BATCH), jnp.float32),
        compiler_params=pltpu.CompilerParams(
            use_tc_tiling_on_sc=True, needs_layout_passes=False
        ),
        scratch_types=(
            [pltpu.VMEM((SEQ, BBLK), jnp.int32)]
            + [pltpu.VMEM((BBLK, DIM), jnp.float32) for _ in range(NBUF)]
            + [pltpu.VMEM((DIM, BBLK), jnp.float32) for _ in range(NBUF)]
            + [pltpu.SemaphoreType.DMA for _ in range(NBUF)]
            + [pltpu.SemaphoreType.DMA for _ in range(NBUF)]
        ),
    )
    def body(tok_hbm, table_hbm, out_hbm, tokv, *rest):
        dstb = rest[:NBUF]
        outb = rest[NBUF:2 * NBUF]
        gsems = rest[2 * NBUF:3 * NBUF]
        osems = rest[3 * NBUF:]
        wid = lax.axis_index("s") * 2 + lax.axis_index("c")
        b0 = wid * BBLK

        pltpu.sync_copy(tok_hbm.at[:, pl.ds(b0, BBLK)], tokv)

        def fire(s, b):
            def gloop(g, c):
                t16 = tokv[s, pl.ds(g * L, L)]
                q16 = lax.shift_right_logical(t16, 3)
                r16 = lax.bitwise_and(t16, 7)
                for u in range(L):
                    pltpu.async_copy(
                        table_hbm.at[q16[u], r16[u]],
                        dstb[b].at[g * L + u],
                        gsems[b],
                    )
                return c

            lax.fori_loop(0, NG, gloop, 0)

        for b in range(NBUF):
            fire(b, b)

        iot = lax.iota(jnp.int32, L)
        jv = [iot + g * L for g in range(NG)]

        def outer(i, carry):
            base = i * NBUF
            for b in range(NBUF):
                s = base + b
                pltpu.make_async_copy(
                    table_hbm.at[0], dstb[b], gsems[b]
                ).wait()
                @pl.when(s >= NBUF)
                def _():
                    pltpu.make_async_copy(
                        outb[b], out_hbm.at[s, :, pl.ds(b0, BBLK)], osems[b]
                    ).wait()

                @plsc.parallel_loop(0, L, step=1, unroll=4)
                def _(c, _b=b):
                    dmod = lax.bitwise_and(iot + c, L - 1)
                    for g in range(NG):
                        for k in range(ND):
                            d0 = k * L
                            plsc.store_scatter(
                                outb[_b],
                                [dmod + d0, jv[g]],
                                plsc.load_gather(
                                    dstb[_b], [jv[g], dmod + d0]
                                ),
                            )

                pltpu.async_copy(
                    outb[b], out_hbm.at[s, :, pl.ds(b0, BBLK)], osems[b]
                )

                @pl.when(s + NBUF < SEQ)
                def _():
                    fire(s + NBUF, b)

            return carry

        lax.fori_loop(0, SEQ // NBUF, outer, 0)

        for k in range(NBUF):
            s = SEQ - NBUF + k
            b = s % NBUF
            pltpu.make_async_copy(
                outb[b], out_hbm.at[s, :, pl.ds(b0, BBLK)], osems[b]
            ).wait()

    return body(tok_t, table3)


def kernel(token_ids, embeddings):
    tok_t = token_ids.T.astype(jnp.int32)
    table3 = embeddings.reshape(NUM_EMB // 8, 8, DIM)
    out_t = _gather_sc(tok_t, table3)
    return out_t.transpose(2, 0, 1)


# speedup vs baseline: 4.9763x; 1.0466x over previous
import functools

import jax
import jax.numpy as jnp
from jax import lax
from jax.experimental import pallas as pl
from jax.experimental.pallas import tpu as pltpu
from jax.experimental.pallas import tpu_sc as plsc

NUM_EMB = 1000000
DIM = 64
BATCH = 4096
SEQ = 200

BBLK = 128
NW = 32
NBUF = 4
L = 16
NG = BBLK // L
ND = DIM // L


def _gather_sc(tok_t, table3):
    mesh = plsc.VectorSubcoreMesh(core_axis_name="c", subcore_axis_name="s")

    @functools.partial(
        pl.kernel,
        mesh=mesh,
        out_type=jax.ShapeDtypeStruct((SEQ, DIM, BATCH), jnp.float32),
        compiler_params=pltpu.CompilerParams(
            use_tc_tiling_on_sc=True, needs_layout_passes=False
        ),
        scratch_types=(
            [pltpu.VMEM((SEQ, BBLK), jnp.int32)]
            + [pltpu.VMEM((BBLK, DIM), jnp.float32) for _ in range(NBUF)]
            + [pltpu.VMEM((DIM, BBLK), jnp.float32) for _ in range(NBUF)]
            + [pltpu.SemaphoreType.DMA for _ in range(NBUF)]
            + [pltpu.SemaphoreType.DMA for _ in range(NBUF)]
        ),
    )
    def body(tok_hbm, table_hbm, out_hbm, tokv, *rest):
        dstb = rest[:NBUF]
        outb = rest[NBUF:2 * NBUF]
        gsems = rest[2 * NBUF:3 * NBUF]
        osems = rest[3 * NBUF:]
        wid = lax.axis_index("s") * 2 + lax.axis_index("c")
        b0 = wid * BBLK

        pltpu.sync_copy(tok_hbm.at[:, pl.ds(b0, BBLK)], tokv)

        def fire(s, b):
            def gloop(g, c):
                t16 = tokv[s, pl.ds(g * L, L)]
                q16 = lax.shift_right_logical(t16, 3)
                r16 = lax.bitwise_and(t16, 7)
                for u in range(L):
                    pltpu.async_copy(
                        table_hbm.at[q16[u], r16[u]],
                        dstb[b].at[g * L + u],
                        gsems[b],
                    )
                return c

            lax.fori_loop(0, NG, gloop, 0)

        for b in range(NBUF):
            fire(b, b)

        iot = lax.iota(jnp.int32, L)
        jv = [iot + g * L for g in range(NG)]

        def outer(i, carry):
            base = i * NBUF
            for b in range(NBUF):
                s = base + b
                pltpu.make_async_copy(
                    table_hbm.at[0], dstb[b], gsems[b]
                ).wait()
                @pl.when(s >= NBUF)
                def _():
                    pltpu.make_async_copy(
                        outb[b], out_hbm.at[s, :, pl.ds(b0, BBLK)], osems[b]
                    ).wait()

                @plsc.parallel_loop(0, L, step=1, unroll=8)
                def _(c, _b=b):
                    dmod = lax.bitwise_and(iot + c, L - 1)
                    for g in range(NG):
                        for k in range(ND):
                            d0 = k * L
                            plsc.store_scatter(
                                outb[_b],
                                [dmod + d0, jv[g]],
                                plsc.load_gather(
                                    dstb[_b], [jv[g], dmod + d0]
                                ),
                            )

                pltpu.async_copy(
                    outb[b], out_hbm.at[s, :, pl.ds(b0, BBLK)], osems[b]
                )

                @pl.when(s + NBUF < SEQ)
                def _():
                    fire(s + NBUF, b)

            return carry

        lax.fori_loop(0, SEQ // NBUF, outer, 0)

        for k in range(NBUF):
            s = SEQ - NBUF + k
            b = s % NBUF
            pltpu.make_async_copy(
                outb[b], out_hbm.at[s, :, pl.ds(b0, BBLK)], osems[b]
            ).wait()

    return body(tok_t, table3)


def kernel(token_ids, embeddings):
    tok_t = token_ids.T.astype(jnp.int32)
    table3 = embeddings.reshape(NUM_EMB // 8, 8, DIM)
    out_t = _gather_sc(tok_t, table3)
    return out_t.transpose(2, 0, 1)
